# baseline (speedup 1.0000x reference)
import functools

import jax
import jax.numpy as jnp
from jax.experimental import pallas as pl


def _rvq_body(num_q, x_ref, cb_ref, out_ref, idx_ref):
    resid = x_ref[...]
    acc = jnp.zeros_like(resid)
    t, d = resid.shape
    k = cb_ref.shape[1]
    iota_k = jax.lax.broadcasted_iota(jnp.int32, (t, k), 1)
    for q in range(num_q):
        cb = cb_ref[q]
        r2 = jnp.sum(resid * resid, axis=-1, keepdims=True)
        c2 = jnp.sum(cb * cb, axis=-1)
        dots = jax.lax.dot_general(
            resid, cb, (((1,), (1,)), ((), ())),
            preferred_element_type=jnp.float32)
        dist = r2 - 2.0 * dots + c2[None, :]
        idx = jnp.argmin(dist, axis=-1).astype(jnp.int32)
        onehot = (iota_k == idx[:, None]).astype(jnp.float32)
        quant = jax.lax.dot_general(
            onehot, cb, (((1,), (0,)), ((), ())),
            preferred_element_type=jnp.float32,
            precision=jax.lax.Precision.HIGHEST)
        s = resid + (quant - resid)
        acc = acc + s
        resid = resid - s
        idx_ref[q, :] = idx
    out_ref[...] = acc


def kernel(x, codebooks):
    b, n, d = x.shape
    num_q, k, _ = codebooks.shape
    tokens = b * n
    t = 256
    xf = x.reshape(tokens, d)
    grid = (tokens // t,)
    out, idx = pl.pallas_call(
        functools.partial(_rvq_body, num_q),
        grid=grid,
        in_specs=[
            pl.BlockSpec((t, d), lambda i: (i, 0)),
            pl.BlockSpec((num_q, k, d), lambda i: (0, 0, 0)),
        ],
        out_specs=[
            pl.BlockSpec((t, d), lambda i: (i, 0)),
            pl.BlockSpec((num_q, t), lambda i: (0, i)),
        ],
        out_shape=[
            jax.ShapeDtypeStruct((tokens, d), jnp.float32),
            jax.ShapeDtypeStruct((num_q, tokens), jnp.int32),
        ],
    )(xf, codebooks)
    return out.reshape(b, n, d), idx.T.reshape(b, n, num_q)


# speedup vs baseline: 3.6441x; 3.6441x over previous
import functools

import jax
import jax.numpy as jnp
from jax.experimental import pallas as pl


def _rvq_body(num_q, x_ref, cb_ref, cbp_ref, out_ref, idx_ref):
    resid = x_ref[...]
    acc = jnp.zeros_like(resid)
    t, d = resid.shape
    k = cb_ref.shape[1]
    iota_k = jax.lax.broadcasted_iota(jnp.int32, (t, k), 1)
    for q in range(num_q):
        cb = cb_ref[q]
        r2 = jnp.sum(resid * resid, axis=-1, keepdims=True)
        c2 = jnp.sum(cb * cb, axis=-1)
        ndots2 = jax.lax.dot_general(
            resid * -2.0, cb, (((1,), (1,)), ((), ())),
            preferred_element_type=jnp.float32)
        dist = r2 + ndots2 + c2[None, :]
        idx = jnp.argmin(dist, axis=-1).astype(jnp.int32)
        onehot = (iota_k == idx[:, None]).astype(jnp.bfloat16)
        g = jax.lax.dot_general(
            onehot, cbp_ref[q], (((1,), (0,)), ((), ())),
            preferred_element_type=jnp.float32)
        quant = (g[:, :d] + g[:, d:2 * d]) + g[:, 2 * d:]
        s = resid + (quant - resid)
        acc = acc + s
        resid = resid - s
        idx_ref[q, :] = idx
    out_ref[...] = acc


def kernel(x, codebooks):
    b, n, d = x.shape
    num_q, k, _ = codebooks.shape
    tokens = b * n
    t = 256
    xf = x.reshape(tokens, d)
    hi = codebooks.astype(jnp.bfloat16)
    rem = codebooks - hi.astype(jnp.float32)
    mid = rem.astype(jnp.bfloat16)
    lo = (rem - mid.astype(jnp.float32)).astype(jnp.bfloat16)
    cb_packed = jnp.concatenate([hi, mid, lo], axis=-1)
    grid = (tokens // t,)
    out, idx = pl.pallas_call(
        functools.partial(_rvq_body, num_q),
        grid=grid,
        in_specs=[
            pl.BlockSpec((t, d), lambda i: (i, 0)),
            pl.BlockSpec((num_q, k, d), lambda i: (0, 0, 0)),
            pl.BlockSpec((num_q, k, 3 * d), lambda i: (0, 0, 0)),
        ],
        out_specs=[
            pl.BlockSpec((t, d), lambda i: (i, 0)),
            pl.BlockSpec((num_q, t), lambda i: (0, i)),
        ],
        out_shape=[
            jax.ShapeDtypeStruct((tokens, d), jnp.float32),
            jax.ShapeDtypeStruct((num_q, tokens), jnp.int32),
        ],
    )(xf, codebooks, cb_packed)
    return out.reshape(b, n, d), idx.T.reshape(b, n, num_q)
